# baseline (speedup 1.0000x reference)
import functools

import jax
import jax.numpy as jnp
from jax import lax
from jax.experimental import pallas as pl
from jax.experimental.pallas import tpu as pltpu
from jax.experimental.pallas import tpu_sc as plsc


def _build(n_rows, n_cols):
    info = plsc.get_sparse_core_info()
    NC, NS, L = info.num_cores, info.num_subcores, info.num_lanes
    NW = NC * NS
    rows_per_w = n_rows // NW
    R = 16
    H = R // 2
    n_chunks = rows_per_w // R
    n_grp = n_cols // L

    mesh = plsc.VectorSubcoreMesh(core_axis_name="c", subcore_axis_name="s")

    @functools.partial(
        pl.kernel,
        mesh=mesh,
        out_type=jax.ShapeDtypeStruct((n_rows, n_cols), jnp.float32),
        compiler_params=pltpu.CompilerParams(needs_layout_passes=False),
        scratch_types=[
            pltpu.VMEM((n_cols,), jnp.int32),
            pltpu.VMEM((R, n_cols), jnp.float32),
            pltpu.VMEM((R, n_cols), jnp.float32),
            pltpu.VMEM((H, n_cols), jnp.float32),
            pltpu.VMEM((H, n_cols), jnp.float32),
            pltpu.SemaphoreType.DMA,
            pltpu.SemaphoreType.DMA,
            pltpu.SemaphoreType.DMA,
            pltpu.SemaphoreType.DMA,
        ],
    )
    def k(x_hbm, perm_hbm, out_hbm, perm_v, i0, i1, o0, o1, si0, si1, so0, so1):
        wid = lax.axis_index("s") * NC + lax.axis_index("c")
        row0 = wid * rows_per_w
        pltpu.sync_copy(perm_hbm, perm_v)
        lane = lax.iota(jnp.int32, L)

        ibufs = (i0, i1)
        obufs = (o0, o1)
        isems = (si0, si1)
        osems = (so0, so1)

        def start_in(ch, b):
            pltpu.async_copy(x_hbm.at[pl.ds(row0 + ch * R, R)], ibufs[b], isems[b])

        def permute_half(ib, ob, h):
            @plsc.parallel_loop(0, n_grp, 1, unroll=2)
            def _(j):
                pidx = perm_v[pl.ds(j * L, L)]
                out_lane = lane + j * L
                vals = [
                    plsc.load_gather(ib, [jnp.full((L,), h * H + r, jnp.int32), pidx])
                    for r in range(H)
                ]
                for r in range(H):
                    ridx = jnp.full((L,), r, jnp.int32)
                    plsc.store_scatter(ob, [ridx, out_lane], vals[r])

        start_in(0, 0)
        start_in(1, 1)

        def outer(c2, carry):
            for b in range(2):
                ch = c2 * 2 + b
                pltpu.make_async_copy(x_hbm.at[pl.ds(0, R)], ibufs[b], isems[b]).wait()

                for h in range(2):
                    if b > 0:
                        pltpu.make_async_copy(
                            obufs[h], out_hbm.at[pl.ds(0, H)], osems[h]
                        ).wait()
                    else:

                        @pl.when(c2 > 0)
                        def _():
                            pltpu.make_async_copy(
                                obufs[h], out_hbm.at[pl.ds(0, H)], osems[h]
                            ).wait()

                    permute_half(ibufs[b], obufs[h], h)
                    pltpu.async_copy(
                        obufs[h],
                        out_hbm.at[pl.ds(row0 + ch * R + h * H, H)],
                        osems[h],
                    )

                @pl.when(ch + 2 < n_chunks)
                def _():
                    start_in(ch + 2, b)

            return carry

        lax.fori_loop(0, n_chunks // 2, outer, 0)

        for h in range(2):
            pltpu.make_async_copy(obufs[h], out_hbm.at[pl.ds(0, H)], osems[h]).wait()

    return k


def kernel(x, perm):
    n_rows, n_cols = x.shape
    out = _build(n_rows, n_cols)(x, perm)
    return (out, 0.0)
